# baseline (speedup 1.0000x reference)
import jax
import jax.numpy as jnp
from jax.experimental import pallas as pl


def _cumprod_block(x_ref, o_ref):
    x = x_ref[...]
    n = x.shape[-1]
    s = 1
    while s < n:
        ones = jnp.ones(x.shape[:-1] + (s,), dtype=x.dtype)
        x = x * jnp.concatenate([ones, x[:, :-s]], axis=-1)
        s *= 2
    o_ref[...] = x


def kernel(x):
    m, n = x.shape
    block_rows = 1024
    return pl.pallas_call(
        _cumprod_block,
        out_shape=jax.ShapeDtypeStruct((m, n), x.dtype),
        grid=(m // block_rows,),
        in_specs=[pl.BlockSpec((block_rows, n), lambda i: (i, 0))],
        out_specs=pl.BlockSpec((block_rows, n), lambda i: (i, 0)),
    )(x)
